# baseline (speedup 1.0000x reference)
import functools

import jax
import jax.numpy as jnp
from jax import lax
from jax.experimental import pallas as pl
from jax.experimental.pallas import tpu as pltpu
from jax.experimental.pallas import tpu_sc as plsc

_NUM_FIELDS = 26
_PER_FIELD = 100000
_EMB_DIM = 32
_NW = 32
_ROWS_PER_DMA = 128
_LANES = 16


def kernel(x, table):
    B, F = x.shape
    N = B * F
    per_w = N // _NW
    n_dma = per_w // _ROWS_PER_DMA

    mesh = plsc.VectorSubcoreMesh(core_axis_name="c", subcore_axis_name="s")

    @functools.partial(
        pl.kernel,
        out_type=jax.ShapeDtypeStruct((N, _EMB_DIM), jnp.float32),
        mesh=mesh,
        scratch_types=[
            pltpu.VMEM((per_w,), jnp.int32),
            pltpu.VMEM((_ROWS_PER_DMA, _EMB_DIM), jnp.float32),
            pltpu.SemaphoreType.DMA,
        ],
        compiler_params=pltpu.CompilerParams(use_tc_tiling_on_sc=False),
    )
    def _emb(x_hbm, table_hbm, out_hbm, idx_v, rows_v, sem):
        wid = lax.axis_index("s") * 2 + lax.axis_index("c")
        base = pl.multiple_of(wid * per_w, 8)
        pltpu.sync_copy(x_hbm.at[pl.ds(base, per_w)], idx_v)

        lane = lax.iota(jnp.int32, _LANES)

        def add_body(i, carry):
            col = pl.multiple_of(i * _LANES, _LANES)
            f = lax.rem(col + lane, _NUM_FIELDS)
            idx_v[pl.ds(col, _LANES)] = idx_v[pl.ds(col, _LANES)] + f * _PER_FIELD
            return carry

        lax.fori_loop(0, per_w // _LANES, add_body, 0)

        def dma_body(r, carry):
            off = pl.multiple_of(r * _ROWS_PER_DMA, 8)
            idx_slice = idx_v.at[pl.ds(off, _ROWS_PER_DMA)]
            pltpu.async_copy(table_hbm.at[idx_slice], rows_v, sem).wait()
            out_off = pl.multiple_of(base + off, 8)
            pltpu.sync_copy(rows_v, out_hbm.at[pl.ds(out_off, _ROWS_PER_DMA)])
            return carry

        lax.fori_loop(0, n_dma, dma_body, 0)

    out = _emb(x.reshape(N), table)
    return out.reshape(B, F, _EMB_DIM)


# speedup vs baseline: 1.0483x; 1.0483x over previous
import functools

import jax
import jax.numpy as jnp
from jax import lax
from jax.experimental import pallas as pl
from jax.experimental.pallas import tpu as pltpu
from jax.experimental.pallas import tpu_sc as plsc

_NUM_FIELDS = 26
_PER_FIELD = 100000
_EMB_DIM = 32
_NW = 32
_ROWS_PER_DMA = 128
_LANES = 16
_NBUF = 8


def kernel(x, table):
    B, F = x.shape
    N = B * F
    per_w = N // _NW
    n_dma = per_w // _ROWS_PER_DMA
    ngroups = n_dma // _NBUF

    mesh = plsc.VectorSubcoreMesh(core_axis_name="c", subcore_axis_name="s")

    @functools.partial(
        pl.kernel,
        out_type=jax.ShapeDtypeStruct((N, _EMB_DIM), jnp.float32),
        mesh=mesh,
        scratch_types=(
            [pltpu.VMEM((per_w,), jnp.int32)]
            + [pltpu.VMEM((_ROWS_PER_DMA, _EMB_DIM), jnp.float32)] * _NBUF
            + [pltpu.SemaphoreType.DMA] * (2 * _NBUF)
        ),
        compiler_params=pltpu.CompilerParams(use_tc_tiling_on_sc=False),
    )
    def _emb(x_hbm, table_hbm, out_hbm, idx_v, *bufs):
        rows = bufs[:_NBUF]
        gsem = bufs[_NBUF:2 * _NBUF]
        wsem = bufs[2 * _NBUF:]

        wid = lax.axis_index("s") * 2 + lax.axis_index("c")
        base = pl.multiple_of(wid * per_w, 8)
        pltpu.sync_copy(x_hbm.at[pl.ds(base, per_w)], idx_v)

        lane = lax.iota(jnp.int32, _LANES)

        def add_body(i, carry):
            col = pl.multiple_of(i * _LANES, _LANES)
            f = lax.rem(col + lane, _NUM_FIELDS)
            idx_v[pl.ds(col, _LANES)] = idx_v[pl.ds(col, _LANES)] + f * _PER_FIELD
            return carry

        lax.fori_loop(0, per_w // _LANES, add_body, 0)

        def gather_desc(r, b):
            off = pl.multiple_of(r * _ROWS_PER_DMA, 8)
            idx_slice = idx_v.at[pl.ds(off, _ROWS_PER_DMA)]
            return pltpu.make_async_copy(table_hbm.at[idx_slice], rows[b], gsem[b])

        def write_desc(r, b):
            out_off = pl.multiple_of(base + r * _ROWS_PER_DMA, 8)
            return pltpu.make_async_copy(
                rows[b], out_hbm.at[pl.ds(out_off, _ROWS_PER_DMA)], wsem[b])

        for b in range(_NBUF):
            gather_desc(b, b).start()

        def group_body(g, carry):
            rbase = g * _NBUF
            for b in range(_NBUF):
                gather_desc(rbase + b, b).wait()
                write_desc(rbase + b, b).start()
            for b in range(_NBUF):
                write_desc(rbase + b, b).wait()

                @pl.when(g < ngroups - 1)
                def _():
                    gather_desc(rbase + _NBUF + b, b).start()

            return carry

        lax.fori_loop(0, ngroups, group_body, 0)

    out = _emb(x.reshape(N), table)
    return out.reshape(B, F, _EMB_DIM)
